# baseline (speedup 1.0000x reference)
import jax
import jax.numpy as jnp
from jax import lax
from jax.experimental import pallas as pl
from jax.experimental.pallas import tpu as pltpu
from jax.experimental.pallas import tpu_sc as plsc

_L = 16
_R = 128
_RTC = 96
_RSC = _R - _RTC
_N = 32768
_NW = 32
_RPW = _RSC // _NW
_CPR = 2
_CH = _N // _CPR
_SEGC = _CH // _L
_P = 8
_SP = _SEGC + _P
_NQ = _RPW * _CPR
_VPS = _SEGC // _L


def _rev_cumsum_body(x_hbm, out_hbm, xin0, xin1, xout0, xout1, xpad, opad,
                     sem_in0, sem_in1, sem_out0, sem_out1):
    cid = lax.axis_index("c")
    sid = lax.axis_index("s")
    wid = sid * 2 + cid
    seg_base = lax.iota(jnp.int32, _L) * _SP
    lane = lax.iota(jnp.int32, _L)
    xins = (xin0, xin1)
    xouts = (xout0, xout1)
    sems_in = (sem_in0, sem_in1)
    sems_out = (sem_out0, sem_out1)

    def start_in(q):
        r, c = divmod(q, _CPR)
        row = _RTC + wid * _RPW + r
        col0 = _N - (c + 1) * _CH
        return pltpu.async_copy(
            x_hbm.at[row, pl.ds(col0, _CH)], xins[q % 2], sems_in[q % 2])

    def start_out(q):
        r, c = divmod(q, _CPR)
        row = wid * _RPW + r
        return pltpu.async_copy(
            xouts[q % 2], out_hbm.at[row, pl.ds(c * _CH, _CH)],
            sems_out[q % 2])

    U = 8
    _W = 15 * _SP + U
    g3c = [jnp.full((_L,), 15 * _SP + U - 1 - k, jnp.int32) - seg_base
           for k in range(U)]
    s3c = [seg_base + k for k in range(U)]

    def compute(xin_b, xout_b, carry):
        def seg1(s, t):
            def p1(i, part):
                v = [None] * 4
                for k in range(4):
                    off = i * _L + k * _L
                    v[k] = xin_b[pl.ds(s * _SEGC + off, _L)]
                    xpad[pl.ds(s * _SP + off, _L)] = v[k]
                return part + ((v[0] + v[1]) + (v[2] + v[3]))

            part = plsc.parallel_loop(
                0, _VPS, step=4, unroll=2,
                carry=jnp.zeros((_L,), jnp.float32))(p1)
            return t + jnp.where(lane == (_L - 1 - s),
                                 jnp.full((_L,), jnp.sum(part)),
                                 jnp.zeros((_L,), jnp.float32))

        t = lax.fori_loop(0, _L, seg1, jnp.zeros((_L,), jnp.float32))

        offs = plsc.cumsum(t) - t + carry
        total = jnp.sum(t)

        def p3(i, acc):
            gwin = xpad.at[pl.ds(_SEGC - U - i, _W)]
            swin = opad.at[pl.ds(i, _W)]
            g = [plsc.load_gather(gwin, [g3c[k]]) for k in range(U)]
            h01 = g[0] + g[1]
            h23 = g[2] + g[3]
            h45 = g[4] + g[5]
            h67 = g[6] + g[7]
            q03 = h01 + h23
            q47 = h45 + h67
            p = [g[0], h01, h01 + g[2], q03, q03 + g[4], q03 + h45,
                 (q03 + h45) + g[6], q03 + q47]
            o = [acc + p[k] for k in range(U)]
            for k in range(U):
                plsc.store_scatter(swin, [s3c[k]], o[k])
            return o[U - 1]

        plsc.parallel_loop(0, _SEGC, step=U, unroll=2, carry=offs)(p3)

        def seg4(s, z):
            def p4(i, zz):
                off = i * _L
                xout_b[pl.ds(s * _SEGC + off, _L)] = (
                    opad[pl.ds(s * _SP + off, _L)])
                return zz

            return plsc.parallel_loop(
                0, _VPS, unroll=8, carry=z)(p4)

        lax.fori_loop(0, _L, seg4, jnp.int32(0))

        return carry + total

    h_in = [None] * _NQ
    h_out = [None] * _NQ
    h_in[0] = start_in(0)
    carry = jnp.float32(0.0)
    for q in range(_NQ):
        if q + 1 < _NQ:
            h_in[q + 1] = start_in(q + 1)
        h_in[q].wait()
        if q >= 2:
            h_out[q - 2].wait()
        if q % _CPR == 0:
            carry = jnp.float32(0.0)
        carry = compute(xins[q % 2], xouts[q % 2], carry)
        h_out[q] = start_out(q)
    h_out[_NQ - 2].wait()
    h_out[_NQ - 1].wait()


def _tc_body(x_ref, o_ref, carry_ref):
    j = pl.program_id(0)

    @pl.when(j == 0)
    def _():
        carry_ref[...] = jnp.zeros_like(carry_ref)

    xb = x_ref[...]
    ia = lax.broadcasted_iota(jnp.int32, (_B, _B), 0)
    ib = lax.broadcasted_iota(jnp.int32, (_B, _B), 1)
    m = (ia + ib >= _B - 1).astype(jnp.float32)
    y = jax.lax.dot(xb, m, preferred_element_type=jnp.float32)
    y = y + carry_ref[...]
    o_ref[...] = y
    carry_ref[...] = y[:, _B - 1:_B]


_B = 256


def _tc_rev_cumsum(x_rows):
    nb = _N // _B
    return pl.pallas_call(
        _tc_body,
        grid=(nb,),
        in_specs=[pl.BlockSpec((_RTC, _B), lambda j: (0, nb - 1 - j))],
        out_specs=pl.BlockSpec((_RTC, _B), lambda j: (0, j)),
        out_shape=jax.ShapeDtypeStruct((_RTC, _N), jnp.float32),
        scratch_shapes=[pltpu.VMEM((_RTC, 1), jnp.float32)],
    )(x_rows)


def kernel(x):
    mesh = plsc.VectorSubcoreMesh(core_axis_name="c", subcore_axis_name="s")
    f = pl.kernel(
        _rev_cumsum_body,
        out_type=jax.ShapeDtypeStruct((_RSC, _N), jnp.float32),
        mesh=mesh,
        scratch_types=[
            pltpu.VMEM((_CH,), jnp.float32),
            pltpu.VMEM((_CH,), jnp.float32),
            pltpu.VMEM((_CH,), jnp.float32),
            pltpu.VMEM((_CH,), jnp.float32),
            pltpu.VMEM((_L * _SP,), jnp.float32),
            pltpu.VMEM((_L * _SP,), jnp.float32),
            pltpu.SemaphoreType.DMA,
            pltpu.SemaphoreType.DMA,
            pltpu.SemaphoreType.DMA,
            pltpu.SemaphoreType.DMA,
        ],
        compiler_params=pltpu.CompilerParams(needs_layout_passes=False),
    )
    out_sc = f(x)
    out_tc = _tc_rev_cumsum(x[:_RTC])
    return jnp.concatenate([out_tc, out_sc], axis=0)


# speedup vs baseline: 2.4855x; 2.4855x over previous
import jax
import jax.numpy as jnp
from jax import lax
from jax.experimental import pallas as pl
from jax.experimental.pallas import tpu as pltpu
from jax.experimental.pallas import tpu_sc as plsc

_L = 16
_R = 128
_N = 32768
_NW = 32
_RPW = _R // _NW
_CPR = 2
_CH = _N // _CPR
_SEGC = _CH // _L
_P = 8
_SP = _SEGC + _P
_NQ = _RPW * _CPR
_VPS = _SEGC // _L


def _rev_cumsum_body(x_hbm, out_hbm, xin0, xin1, xout0, xout1, xpad, opad,
                     sem_in0, sem_in1, sem_out0, sem_out1):
    cid = lax.axis_index("c")
    sid = lax.axis_index("s")
    wid = sid * 2 + cid
    seg_base = lax.iota(jnp.int32, _L) * _SP
    lane = lax.iota(jnp.int32, _L)
    xins = (xin0, xin1)
    xouts = (xout0, xout1)
    sems_in = (sem_in0, sem_in1)
    sems_out = (sem_out0, sem_out1)

    def start_in(q):
        r, c = divmod(q, _CPR)
        row = wid * _RPW + r
        col0 = _N - (c + 1) * _CH
        return pltpu.async_copy(
            x_hbm.at[row, pl.ds(col0, _CH)], xins[q % 2], sems_in[q % 2])

    def start_out(q):
        r, c = divmod(q, _CPR)
        row = wid * _RPW + r
        return pltpu.async_copy(
            xouts[q % 2], out_hbm.at[row, pl.ds(c * _CH, _CH)],
            sems_out[q % 2])

    U = 8
    _W = 15 * _SP + U
    g3c = [jnp.full((_L,), 15 * _SP + U - 1 - k, jnp.int32) - seg_base
           for k in range(U)]
    s3c = [seg_base + k for k in range(U)]

    def compute(xin_b, xout_b, carry):
        def seg1(s, z):
            def p1(i, zz):
                for k in range(8):
                    off = i * _L + k * _L
                    xpad[pl.ds(s * _SP + off, _L)] = (
                        xin_b[pl.ds(s * _SEGC + off, _L)])
                return zz

            return plsc.parallel_loop(0, _VPS, step=8, carry=z)(p1)

        lax.fori_loop(0, _L, seg1, jnp.int32(0))

        def p3(i, acc):
            gwin = xpad.at[pl.ds(_SEGC - U - i, _W)]
            swin = opad.at[pl.ds(i, _W)]
            g = [plsc.load_gather(gwin, [g3c[k]]) for k in range(U)]
            h01 = g[0] + g[1]
            h23 = g[2] + g[3]
            h45 = g[4] + g[5]
            h67 = g[6] + g[7]
            q03 = h01 + h23
            q47 = h45 + h67
            p = [g[0], h01, h01 + g[2], q03, q03 + g[4], q03 + h45,
                 (q03 + h45) + g[6], q03 + q47]
            o = [acc + p[k] for k in range(U)]
            for k in range(U):
                plsc.store_scatter(swin, [s3c[k]], o[k])
            return o[U - 1]

        t = plsc.parallel_loop(
            0, _SEGC, step=U, unroll=2,
            carry=jnp.zeros((_L,), jnp.float32))(p3)

        offs = plsc.cumsum(t) - t + carry
        total = jnp.sum(t)

        def seg4(s, z):
            bias = jnp.full(
                (_L,), jnp.sum(jnp.where(lane == s, offs,
                                         jnp.zeros((_L,), jnp.float32))))

            def p4(i, zz):
                off = i * _L
                xout_b[pl.ds(s * _SEGC + off, _L)] = (
                    opad[pl.ds(s * _SP + off, _L)] + bias)
                return zz

            return plsc.parallel_loop(
                0, _VPS, unroll=16, carry=z)(p4)

        lax.fori_loop(0, _L, seg4, jnp.int32(0))

        return carry + total

    h_in = [None] * _NQ
    h_out = [None] * _NQ
    h_in[0] = start_in(0)
    carry = jnp.float32(0.0)
    for q in range(_NQ):
        if q + 1 < _NQ:
            h_in[q + 1] = start_in(q + 1)
        h_in[q].wait()
        if q >= 2:
            h_out[q - 2].wait()
        if q % _CPR == 0:
            carry = jnp.float32(0.0)
        carry = compute(xins[q % 2], xouts[q % 2], carry)
        h_out[q] = start_out(q)
    h_out[_NQ - 2].wait()
    h_out[_NQ - 1].wait()


def kernel(x):
    mesh = plsc.VectorSubcoreMesh(core_axis_name="c", subcore_axis_name="s")
    f = pl.kernel(
        _rev_cumsum_body,
        out_type=jax.ShapeDtypeStruct((_R, _N), jnp.float32),
        mesh=mesh,
        scratch_types=[
            pltpu.VMEM((_CH,), jnp.float32),
            pltpu.VMEM((_CH,), jnp.float32),
            pltpu.VMEM((_CH,), jnp.float32),
            pltpu.VMEM((_CH,), jnp.float32),
            pltpu.VMEM((_L * _SP,), jnp.float32),
            pltpu.VMEM((_L * _SP,), jnp.float32),
            pltpu.SemaphoreType.DMA,
            pltpu.SemaphoreType.DMA,
            pltpu.SemaphoreType.DMA,
            pltpu.SemaphoreType.DMA,
        ],
        compiler_params=pltpu.CompilerParams(needs_layout_passes=False),
    )
    return f(x)


# speedup vs baseline: 2.5968x; 1.0448x over previous
import jax
import jax.numpy as jnp
from jax import lax
from jax.experimental import pallas as pl
from jax.experimental.pallas import tpu as pltpu
from jax.experimental.pallas import tpu_sc as plsc

_L = 16
_R = 128
_N = 32768
_NW = 32
_RPW = _R // _NW
_CPR = 2
_CH = _N // _CPR
_SEGC = _CH // _L
_P = 8
_SP = _SEGC + _P
_NQ = _RPW * _CPR
_VPS = _SEGC // _L


def _rev_cumsum_body(x_hbm, out_hbm, xin0, xin1, xout0, xout1, xpad, opad,
                     sem_in0, sem_in1, sem_out0, sem_out1):
    cid = lax.axis_index("c")
    sid = lax.axis_index("s")
    wid = sid * 2 + cid
    seg_base = lax.iota(jnp.int32, _L) * _SP
    lane = lax.iota(jnp.int32, _L)
    xins = (xin0, xin1)
    xouts = (xout0, xout1)
    sems_in = (sem_in0, sem_in1)
    sems_out = (sem_out0, sem_out1)

    def start_in(q):
        r, c = divmod(q, _CPR)
        row = wid * _RPW + r
        col0 = _N - (c + 1) * _CH
        return pltpu.async_copy(
            x_hbm.at[row, pl.ds(col0, _CH)], xins[q % 2], sems_in[q % 2])

    def start_out(q):
        r, c = divmod(q, _CPR)
        row = wid * _RPW + r
        return pltpu.async_copy(
            xouts[q % 2], out_hbm.at[row, pl.ds(c * _CH, _CH)],
            sems_out[q % 2])

    U = 8
    _W = 15 * _SP + U
    g3c = [jnp.full((_L,), 15 * _SP + U - 1 - k, jnp.int32) - seg_base
           for k in range(U)]
    s3c = [seg_base + k for k in range(U)]

    def compute(xin_b, xout_b, carry):
        def seg1(s, t):
            def p1(i, part):
                v = [None] * 4
                for k in range(4):
                    off = i * _L + k * _L
                    v[k] = xin_b[pl.ds(s * _SEGC + off, _L)]
                    xpad[pl.ds(s * _SP + off, _L)] = v[k]
                return part + ((v[0] + v[1]) + (v[2] + v[3]))

            part = plsc.parallel_loop(
                0, _VPS, step=4, unroll=2,
                carry=jnp.zeros((_L,), jnp.float32))(p1)
            return t + jnp.where(lane == (_L - 1 - s),
                                 jnp.full((_L,), jnp.sum(part)),
                                 jnp.zeros((_L,), jnp.float32))

        t = lax.fori_loop(0, _L, seg1, jnp.zeros((_L,), jnp.float32))

        offs = plsc.cumsum(t) - t + carry
        total = jnp.sum(t)

        def p3(i, acc):
            gwin = xpad.at[pl.ds(_SEGC - U - i, _W)]
            swin = opad.at[pl.ds(i, _W)]
            g = [plsc.load_gather(gwin, [g3c[k]]) for k in range(U)]
            h01 = g[0] + g[1]
            h23 = g[2] + g[3]
            h45 = g[4] + g[5]
            h67 = g[6] + g[7]
            q03 = h01 + h23
            q47 = h45 + h67
            p = [g[0], h01, h01 + g[2], q03, q03 + g[4], q03 + h45,
                 (q03 + h45) + g[6], q03 + q47]
            o = [acc + p[k] for k in range(U)]
            for k in range(U):
                plsc.store_scatter(swin, [s3c[k]], o[k])
            return o[U - 1]

        plsc.parallel_loop(0, _SEGC, step=U, unroll=2, carry=offs)(p3)

        def seg4(s, z):
            def p4(i, zz):
                off = i * _L
                xout_b[pl.ds(s * _SEGC + off, _L)] = (
                    opad[pl.ds(s * _SP + off, _L)])
                return zz

            return plsc.parallel_loop(
                0, _VPS, unroll=8, carry=z)(p4)

        lax.fori_loop(0, _L, seg4, jnp.int32(0))

        return carry + total

    h_in = [None] * _NQ
    h_out = [None] * _NQ
    h_in[0] = start_in(0)
    carry = jnp.float32(0.0)
    for q in range(_NQ):
        if q + 1 < _NQ:
            h_in[q + 1] = start_in(q + 1)
        h_in[q].wait()
        if q >= 2:
            h_out[q - 2].wait()
        if q % _CPR == 0:
            carry = jnp.float32(0.0)
        carry = compute(xins[q % 2], xouts[q % 2], carry)
        h_out[q] = start_out(q)
    h_out[_NQ - 2].wait()
    h_out[_NQ - 1].wait()


def kernel(x):
    mesh = plsc.VectorSubcoreMesh(core_axis_name="c", subcore_axis_name="s")
    f = pl.kernel(
        _rev_cumsum_body,
        out_type=jax.ShapeDtypeStruct((_R, _N), jnp.float32),
        mesh=mesh,
        scratch_types=[
            pltpu.VMEM((_CH,), jnp.float32),
            pltpu.VMEM((_CH,), jnp.float32),
            pltpu.VMEM((_CH,), jnp.float32),
            pltpu.VMEM((_CH,), jnp.float32),
            pltpu.VMEM((_L * _SP,), jnp.float32),
            pltpu.VMEM((_L * _SP,), jnp.float32),
            pltpu.SemaphoreType.DMA,
            pltpu.SemaphoreType.DMA,
            pltpu.SemaphoreType.DMA,
            pltpu.SemaphoreType.DMA,
        ],
        compiler_params=pltpu.CompilerParams(needs_layout_passes=False),
    )
    return f(x)
